# baseline (speedup 1.0000x reference)
import functools

import jax
import jax.numpy as jnp
from jax.experimental import pallas as pl
from jax.experimental.pallas import tpu as pltpu

MAX_SPAN = 8
NEG = -1e30


def _span_body(cu_ref, hid_ref, twT_ref, win_ref, binT_ref, wscT_ref,
               bsc_ref, out_ref, *, T):
    D = win_ref.shape[0]
    qvT = jax.lax.dot_general(
        twT_ref[:, :], win_ref[:, :], (((1,), (1,)), ((), ())),
        preferred_element_type=jnp.float32) + binT_ref[:, :]
    projT = jnp.concatenate(
        [qvT, wscT_ref[:, :], jnp.zeros((MAX_SPAN - 2, D), jnp.float32)],
        axis=0)
    lcT = jax.lax.dot_general(
        projT, hid_ref[:, :], (((1,), (1,)), ((), ())),
        preferred_element_type=jnp.float32)

    lc_ext = jnp.concatenate([lcT, lcT[:, :MAX_SPAN]], axis=1)
    lT = lc_ext[0:1, :]
    cT = lc_ext[1:2, :]

    LwT = jnp.concatenate([lT[:, w:w + T] for w in range(MAX_SPAN)], axis=0)
    CwT = jnp.concatenate([cT[:, w:w + T] for w in range(MAX_SPAN)], axis=0)

    pos = jax.lax.broadcasted_iota(jnp.int32, (1, T), 1)
    seq_end = jnp.full((1, T), T, jnp.int32)
    for j in range(1, cu_ref.shape[0]):
        b = cu_ref[j]
        seq_end = jnp.minimum(seq_end, jnp.where(b > pos, b, T))
    rem = seq_end - pos

    wrow = jax.lax.broadcasted_iota(jnp.int32, (MAX_SPAN, 1), 0)
    zfull = jnp.where(wrow < rem, LwT, NEG)
    bsc = bsc_ref[0, 0]
    rows = []
    for s in range(MAX_SPAN):
        z = zfull[:s + 1]
        m = jnp.max(z, axis=0, keepdims=True)
        e = jnp.exp(z - m)
        denom = jnp.sum(e, axis=0, keepdims=True)
        num = jnp.sum(e * CwT[:s + 1], axis=0, keepdims=True)
        rows.append(num / denom + bsc)
    res = jnp.concatenate(rows, axis=0)
    out_ref[:, :] = jnp.transpose(
        res.reshape(MAX_SPAN, T // 16, 16), (1, 2, 0)).reshape(T // 16, 128)


@jax.jit
def kernel(hidden, cu_seqlens, termWeight, W_in, b_in, W_score, b_score):
    T, D = hidden.shape
    full = lambda shape: pl.BlockSpec(shape, lambda: (0, 0),
                                      memory_space=pltpu.VMEM)
    out = pl.pallas_call(
        functools.partial(_span_body, T=T),
        out_shape=jax.ShapeDtypeStruct((T // 16, 128), jnp.float32),
        in_specs=[
            pl.BlockSpec(memory_space=pltpu.SMEM),
            full((T, D)),
            full((1, D)),
            full((D, D)),
            full((1, D)),
            full((1, D)),
            full((1, 1)),
        ],
        out_specs=full((T // 16, 128)),
    )(cu_seqlens, hidden, termWeight.reshape(1, D), W_in,
      b_in.reshape(1, D), W_score.reshape(1, D), b_score.reshape(1, 1))
    return out.reshape(T * MAX_SPAN, 1)


# speedup vs baseline: 1.2370x; 1.2370x over previous
import functools

import jax
import jax.numpy as jnp
from jax.experimental import pallas as pl
from jax.experimental.pallas import tpu as pltpu

MAX_SPAN = 8
NEG = -1e30


def _span_body(cu_ref, hid_ref, twT_ref, win_ref, binT_ref, wscT_ref,
               bsc_ref, out_ref, *, T):
    D = win_ref.shape[0]
    qvT = jax.lax.dot_general(
        twT_ref[:, :], win_ref[:, :], (((1,), (1,)), ((), ())),
        preferred_element_type=jnp.float32) + binT_ref[:, :]
    projT = jnp.concatenate(
        [qvT, wscT_ref[:, :], jnp.zeros((MAX_SPAN - 2, D), jnp.float32)],
        axis=0)
    lcT = jax.lax.dot_general(
        projT, hid_ref[:, :], (((1,), (1,)), ((), ())),
        preferred_element_type=jnp.float32)

    lc_ext = jnp.concatenate([lcT, lcT[:, :MAX_SPAN]], axis=1)
    lT = lc_ext[0:1, :]
    cT = lc_ext[1:2, :]

    LwT = jnp.concatenate([lT[:, w:w + T] for w in range(MAX_SPAN)], axis=0)
    CwT = jnp.concatenate([cT[:, w:w + T] for w in range(MAX_SPAN)], axis=0)

    pos = jax.lax.broadcasted_iota(jnp.int32, (1, T), 1)
    seq_end = jnp.full((1, T), T, jnp.int32)
    for j in range(1, cu_ref.shape[0]):
        b = cu_ref[j]
        seq_end = jnp.minimum(seq_end, jnp.where(b > pos, b, T))
    rem = seq_end - pos

    wrow = jax.lax.broadcasted_iota(jnp.int32, (MAX_SPAN, 1), 0)
    zfull = jnp.where(wrow < rem, LwT, NEG)
    bsc = bsc_ref[0, 0]
    rows = []
    for s in range(MAX_SPAN):
        z = zfull[:s + 1]
        m = jnp.max(z, axis=0, keepdims=True)
        e = jnp.exp(z - m)
        denom = jnp.sum(e, axis=0, keepdims=True)
        num = jnp.sum(e * CwT[:s + 1], axis=0, keepdims=True)
        rows.append(num / denom + bsc)
    res = jnp.concatenate(rows, axis=0)
    X = jnp.transpose(
        res.reshape(MAX_SPAN, T // 16, 16), (1, 0, 2)).reshape(T // 16, 128)
    lane = jax.lax.broadcasted_iota(jnp.int32, (1, 128), 1)
    perm = (lane % 8) * 16 + lane // 8
    out_ref[:, :] = jnp.take_along_axis(
        X, jnp.broadcast_to(perm, (T // 16, 128)), axis=1)


@jax.jit
def kernel(hidden, cu_seqlens, termWeight, W_in, b_in, W_score, b_score):
    T, D = hidden.shape
    full = lambda shape: pl.BlockSpec(shape, lambda: (0, 0),
                                      memory_space=pltpu.VMEM)
    out = pl.pallas_call(
        functools.partial(_span_body, T=T),
        out_shape=jax.ShapeDtypeStruct((T // 16, 128), jnp.float32),
        in_specs=[
            pl.BlockSpec(memory_space=pltpu.SMEM),
            full((T, D)),
            full((1, D)),
            full((D, D)),
            full((1, D)),
            full((1, D)),
            full((1, 1)),
        ],
        out_specs=full((T // 16, 128)),
    )(cu_seqlens, hidden, termWeight.reshape(1, D), W_in,
      b_in.reshape(1, D), W_score.reshape(1, D), b_score.reshape(1, 1))
    return out.reshape(T * MAX_SPAN, 1)
